# baseline (speedup 1.0000x reference)
import functools

import jax
import jax.numpy as jnp
from jax import lax
from jax.experimental import pallas as pl
from jax.experimental.pallas import tpu as pltpu
from jax.experimental.pallas import tpu_sc as plsc

_NN = 50000
_NE = 100000
_AD = 32
_BD = 16
_NW = 32
_CH = 128
_E2 = 102400
_SP_ROWS = 51200


_GRP = 5


def _sc_gather(atom, nbr_pad):
    per_w = _E2 // _NW
    n_grp = per_w // (_CH * _GRP)
    mesh = plsc.VectorSubcoreMesh(core_axis_name="c", subcore_axis_name="s")

    @functools.partial(
        pl.kernel,
        mesh=mesh,
        out_type=jax.ShapeDtypeStruct((_E2, _AD), jnp.float32),
        scratch_types=[
            pltpu.VMEM((per_w,), jnp.int32),
            pltpu.VMEM((2, _GRP, _CH, _AD), jnp.float32),
            pltpu.SemaphoreType.DMA,
            pltpu.SemaphoreType.DMA,
        ],
        compiler_params=pltpu.CompilerParams(use_tc_tiling_on_sc=False),
    )
    def k(atom_hbm, idx_hbm, out_hbm, idx_all, bufs, gsem, wsem):
        wid = lax.axis_index("s") * 2 + lax.axis_index("c")
        base = wid * per_w
        pltpu.sync_copy(idx_hbm.at[pl.ds(base, per_w)], idx_all)

        def group(g, carry):
            p = g % 2
            cbase = g * _GRP

            @pl.when(g >= 2)
            def _drain_old():
                for b in range(_GRP):
                    pltpu.make_async_copy(
                        out_hbm.at[pl.ds(0, _CH)], bufs.at[p, b], wsem).wait()

            hs = [
                pltpu.async_copy(
                    atom_hbm.at[idx_all.at[pl.ds((cbase + b) * _CH, _CH)]],
                    bufs.at[p, b], gsem)
                for b in range(_GRP)
            ]
            for h in hs:
                h.wait()
            for b in range(_GRP):
                pltpu.async_copy(
                    bufs.at[p, b],
                    out_hbm.at[pl.ds(base + (cbase + b) * _CH, _CH)], wsem)
            return carry

        lax.fori_loop(0, n_grp, group, 0)
        for _ in range(2 * _GRP):
            pltpu.make_async_copy(
                out_hbm.at[pl.ds(0, _CH)], bufs.at[0, 0], wsem).wait()

    return k(atom, nbr_pad)


def _tc_transform(bond_q, a_pk, wcat_t):
    be = 1024
    rb = be // 4

    def body(bq_ref, a_ref, wt_ref, out_ref):
        p = a_ref[...]
        bq = bq_ref[...]
        cols = []
        for g in range(4):
            pg = p[:, g * _AD:(g + 1) * _AD]
            ht = lax.dot_general(wt_ref[...], pg, (((1,), (1,)), ((), ())),
                                 preferred_element_type=jnp.float32)
            bt = bq[g]
            acc = ht[512:544, :]
            for kk in range(_BD):
                acc = acc + bt[kk:kk + 1, :] * ht[kk * 32:(kk + 1) * 32, :]
            cols.append(acc.T)
        out_ref[...] = jnp.concatenate(cols, axis=1)

    return pl.pallas_call(
        body,
        grid=(_E2 // be,),
        in_specs=[
            pl.BlockSpec((4, _BD, rb), lambda i: (0, 0, i)),
            pl.BlockSpec((rb, 4 * _AD), lambda i: (i, 0)),
            pl.BlockSpec((544, _AD), lambda i: (0, 0)),
        ],
        out_specs=pl.BlockSpec((rb, 4 * _AD), lambda i: (i, 0)),
        out_shape=jax.ShapeDtypeStruct((_E2 // 4, 4 * _AD), jnp.float32),
    )(bond_q, a_pk, wcat_t)


_SPLIT = 25600
_SP2_ROWS = 26624
_DUMP = 26000


def _sc_segment_sum(transformed, src2):
    per_t = _E2 // 16
    n_grp = per_t // (_CH * _GRP)
    zrows = _SP2_ROWS // 16
    mesh = plsc.VectorSubcoreMesh(core_axis_name="c", subcore_axis_name="s")

    @functools.partial(
        pl.kernel,
        mesh=mesh,
        out_type=jax.ShapeDtypeStruct((_NN, _AD), jnp.float32),
        scratch_types=[
            pltpu.VMEM((2, _GRP, _CH), jnp.int32),
            pltpu.VMEM((2, _GRP, _CH, _AD), jnp.float32),
            pltpu.VMEM((_CH, _AD), jnp.float32),
            pltpu.VMEM_SHARED((_SP2_ROWS, _AD), jnp.float32),
            pltpu.SemaphoreType.DMA,
            pltpu.SemaphoreType.DMA,
        ],
        compiler_params=pltpu.CompilerParams(use_tc_tiling_on_sc=False),
    )
    def k(t_hbm, src_hbm, out_hbm, idxm, bufs, zbuf, acc_sp, lsem, ssem):
        cid = lax.axis_index("c")
        sid = lax.axis_index("s")
        lo = cid * _SPLIT
        hi = jnp.where(cid == 0, _SPLIT, _NN)

        zero16 = jnp.zeros((16,), jnp.float32)

        def zb(r, carry):
            zbuf[r, 0:16] = zero16
            zbuf[r, 16:32] = zero16
            return carry

        lax.fori_loop(0, _CH, zb, 0)

        def zs(cnk, carry):
            pltpu.sync_copy(
                zbuf, acc_sp.at[pl.ds(sid * zrows + cnk * _CH, _CH)])
            return carry

        lax.fori_loop(0, zrows // _CH, zs, 0)

        plsc.subcore_barrier()

        base = sid * per_t

        def group(g, carry):
            p = g % 2
            cbase = g * _GRP

            pltpu.sync_copy(
                src_hbm.at[pl.ds(sid * (per_t // _CH) + cbase, _GRP)],
                idxm.at[p])
            hs = [
                pltpu.async_copy(
                    t_hbm.at[pl.ds(base + (cbase + b) * _CH, _CH)],
                    bufs.at[p, b], lsem)
                for b in range(_GRP)
            ]
            for b in range(_GRP):
                for i in range(_CH // 16):
                    v = idxm[p, b, pl.ds(i * 16, 16)]
                    ok = (v >= lo) & (v < hi)
                    idxm[p, b, pl.ds(i * 16, 16)] = jnp.where(
                        ok, v - lo, jnp.int32(_DUMP))
            for h in hs:
                h.wait()
            ws = [
                pltpu.async_copy(
                    bufs.at[p, b], acc_sp.at[idxm.at[p, b]], ssem, add=True)
                for b in range(_GRP)
            ]
            for w in ws:
                w.wait()
            return carry

        lax.fori_loop(0, n_grp, group, 0)

        plsc.subcore_barrier()

        @pl.when(cid == 0)
        def _flush0():
            n = _SPLIT // 16
            pltpu.sync_copy(acc_sp.at[pl.ds(sid * n, n)],
                            out_hbm.at[pl.ds(sid * n, n)])

        @pl.when(cid == 1)
        def _flush1():
            n = (_NN - _SPLIT) // 16
            pltpu.sync_copy(acc_sp.at[pl.ds(sid * n, n)],
                            out_hbm.at[pl.ds(_SPLIT + sid * n, n)])

    return k(transformed, src2)


def kernel(atom_features, bond_features, pair_indices, kernel, bias):
    w_aug = jnp.concatenate([kernel, bias[None, :]], axis=0)
    wcat_t = w_aug.reshape(17 * _AD, _AD)

    pad = _E2 - _NE
    nbr_pad = jnp.concatenate(
        [pair_indices[:, 1], jnp.zeros((pad,), jnp.int32)])
    src2 = jnp.concatenate(
        [pair_indices[:, 0], jnp.full((pad,), _NN, jnp.int32)]
    ).reshape(_E2 // _CH, _CH)
    bond_q = jnp.concatenate(
        [bond_features, jnp.zeros((pad, _BD), jnp.float32)]
    ).reshape(_E2 // 4, 4, _BD).transpose(1, 2, 0)

    a_nbr = _sc_gather(atom_features, nbr_pad)
    a_pk = a_nbr.reshape(_E2 // 4, 4 * _AD)
    t_pk = _tc_transform(bond_q, a_pk, wcat_t)
    transformed = t_pk.reshape(_E2, _AD)
    return _sc_segment_sum(transformed, src2)


# speedup vs baseline: 1.0928x; 1.0928x over previous
import functools

import jax
import jax.numpy as jnp
from jax import lax
from jax.experimental import pallas as pl
from jax.experimental.pallas import tpu as pltpu
from jax.experimental.pallas import tpu_sc as plsc

_NN = 50000
_NE = 100000
_AD = 32
_BD = 16
_NW = 32
_CH = 128
_E2 = 102400
_SP_ROWS = 51200


_GRP = 5


def _sc_gather(atom, nbr_pad):
    per_w = _E2 // _NW
    n_grp = per_w // (_CH * _GRP)
    mesh = plsc.VectorSubcoreMesh(core_axis_name="c", subcore_axis_name="s")

    @functools.partial(
        pl.kernel,
        mesh=mesh,
        out_type=jax.ShapeDtypeStruct((_E2, _AD), jnp.float32),
        scratch_types=[
            pltpu.VMEM((per_w,), jnp.int32),
            pltpu.VMEM((_GRP, _CH, _AD), jnp.float32),
            pltpu.SemaphoreType.DMA,
            pltpu.SemaphoreType.DMA,
        ],
        compiler_params=pltpu.CompilerParams(use_tc_tiling_on_sc=False),
    )
    def k(atom_hbm, idx_hbm, out_hbm, idx_all, bufs, gsem, wsem):
        wid = lax.axis_index("s") * 2 + lax.axis_index("c")
        base = wid * per_w
        pltpu.sync_copy(idx_hbm.at[pl.ds(base, per_w)], idx_all)

        def group(g, carry):
            cbase = g * _GRP
            hs = [
                pltpu.async_copy(
                    atom_hbm.at[idx_all.at[pl.ds((cbase + b) * _CH, _CH)]],
                    bufs.at[b], gsem)
                for b in range(_GRP)
            ]
            for h in hs:
                h.wait()
            ws = [
                pltpu.async_copy(
                    bufs.at[b],
                    out_hbm.at[pl.ds(base + (cbase + b) * _CH, _CH)], wsem)
                for b in range(_GRP)
            ]
            for w in ws:
                w.wait()
            return carry

        lax.fori_loop(0, n_grp, group, 0)

    return k(atom, nbr_pad)


def _tc_transform(bond_q, a_pk, wcat_t):
    be = 1024
    rb = be // 4

    def body(bq_ref, a_ref, wt_ref, out_ref):
        p = a_ref[...]
        bq = bq_ref[...]
        cols = []
        for g in range(4):
            pg = p[:, g * _AD:(g + 1) * _AD]
            ht = lax.dot_general(wt_ref[...], pg, (((1,), (1,)), ((), ())),
                                 preferred_element_type=jnp.float32)
            bt = bq[g]
            acc = ht[512:544, :]
            for kk in range(_BD):
                acc = acc + bt[kk:kk + 1, :] * ht[kk * 32:(kk + 1) * 32, :]
            cols.append(acc.T)
        out_ref[...] = jnp.concatenate(cols, axis=1)

    return pl.pallas_call(
        body,
        grid=(_E2 // be,),
        in_specs=[
            pl.BlockSpec((4, _BD, rb), lambda i: (0, 0, i)),
            pl.BlockSpec((rb, 4 * _AD), lambda i: (i, 0)),
            pl.BlockSpec((544, _AD), lambda i: (0, 0)),
        ],
        out_specs=pl.BlockSpec((rb, 4 * _AD), lambda i: (i, 0)),
        out_shape=jax.ShapeDtypeStruct((_E2 // 4, 4 * _AD), jnp.float32),
    )(bond_q, a_pk, wcat_t)


def _sc_segment_sum(transformed, src2):
    per_t = _E2 // 16
    n_grp = per_t // (_CH * _GRP)
    zrows = _SP_ROWS // 16
    orows = _NN // 16
    mesh = plsc.VectorSubcoreMesh(core_axis_name="c", subcore_axis_name="s")

    @functools.partial(
        pl.kernel,
        mesh=mesh,
        out_type=jax.ShapeDtypeStruct((_NN, _AD), jnp.float32),
        scratch_types=[
            pltpu.VMEM((_GRP, _CH), jnp.int32),
            pltpu.VMEM((_GRP, _CH, _AD), jnp.float32),
            pltpu.VMEM((_CH, _AD), jnp.float32),
            pltpu.VMEM_SHARED((_SP_ROWS, _AD), jnp.float32),
            pltpu.SemaphoreType.DMA,
            pltpu.SemaphoreType.DMA,
        ],
        compiler_params=pltpu.CompilerParams(use_tc_tiling_on_sc=False),
    )
    def k(t_hbm, src_hbm, out_hbm, idx_all, bufs, zbuf, acc_sp, lsem, ssem):
        cid = lax.axis_index("c")
        sid = lax.axis_index("s")

        @pl.when(cid == 0)
        def _zero():
            zero16 = jnp.zeros((16,), jnp.float32)

            def zb(r, carry):
                zbuf[r, 0:16] = zero16
                zbuf[r, 16:32] = zero16
                return carry

            lax.fori_loop(0, _CH, zb, 0)

            def zs(cnk, carry):
                pltpu.sync_copy(
                    zbuf, acc_sp.at[pl.ds(sid * zrows + cnk * _CH, _CH)])
                return carry

            lax.fori_loop(0, zrows // _CH, zs, 0)

        plsc.subcore_barrier()

        @pl.when(cid == 0)
        def _scatter():
            base = sid * per_t

            def group(g, carry):
                cbase = g * _GRP
                pltpu.sync_copy(
                    src_hbm.at[pl.ds(sid * (per_t // _CH) + cbase, _GRP)],
                    idx_all)
                hs = [
                    pltpu.async_copy(
                        t_hbm.at[pl.ds(base + (cbase + b) * _CH, _CH)],
                        bufs.at[b], lsem)
                    for b in range(_GRP)
                ]
                for h in hs:
                    h.wait()
                ws = [
                    pltpu.async_copy(
                        bufs.at[b], acc_sp.at[idx_all.at[b]],
                        ssem, add=True)
                    for b in range(_GRP)
                ]
                for w in ws:
                    w.wait()
                return carry

            lax.fori_loop(0, n_grp, group, 0)

        plsc.subcore_barrier()

        @pl.when(cid == 0)
        def _flush():
            pltpu.sync_copy(acc_sp.at[pl.ds(sid * orows, orows)],
                            out_hbm.at[pl.ds(sid * orows, orows)])

    return k(transformed, src2)


def kernel(atom_features, bond_features, pair_indices, kernel, bias):
    w_aug = jnp.concatenate([kernel, bias[None, :]], axis=0)
    wcat_t = w_aug.reshape(17 * _AD, _AD)

    pad = _E2 - _NE
    nbr_pad = jnp.concatenate(
        [pair_indices[:, 1], jnp.zeros((pad,), jnp.int32)])
    src2 = jnp.concatenate(
        [pair_indices[:, 0], jnp.full((pad,), _NN, jnp.int32)]
    ).reshape(_E2 // _CH, _CH)
    bond_q = jnp.concatenate(
        [bond_features, jnp.zeros((pad, _BD), jnp.float32)]
    ).reshape(_E2 // 4, 4, _BD).transpose(1, 2, 0)

    a_nbr = _sc_gather(atom_features, nbr_pad)
    a_pk = a_nbr.reshape(_E2 // 4, 4 * _AD)
    t_pk = _tc_transform(bond_q, a_pk, wcat_t)
    transformed = t_pk.reshape(_E2, _AD)
    return _sc_segment_sum(transformed, src2)
